# Initial kernel scaffold; baseline (speedup 1.0000x reference)
import jax
import jax.numpy as jnp
from jax.experimental import pallas as pl


def kernel(sup_x, y, edge_index, edge_weight, assign_index, assign_weight, W_bi, W_uni):
    raise NotImplementedError("write your pallas kernel here")


# speedup vs baseline: 18.0143x; 18.0143x over previous
import functools

import dataclasses

import jax
import jax.numpy as jnp
from jax import lax
from jax.experimental import pallas as pl
from jax.experimental.pallas import tpu as pltpu
from jax.experimental.pallas import tpu_sc as plsc

_N1 = 10000
_N2 = 5000
_D = 128
_NC = 2
_NS = 16
_NW = _NC * _NS
_L = 16

_NBLK = 88
_BLK = 128
_CB = 8
_NCHUNK = _NBLK // _CB
_CEDGE = _CB * _BLK
_EPT = _NBLK * _BLK
_ETOT = _NW * _EPT

_ZONE1 = 10240
_TROWS = 16384
_DEGC = _TROWS // _NS
_ACCP = 10240
_ACCC = _ACCP // _NS


def _sc_compiler_params():
    cp = pltpu.CompilerParams()
    if "needs_layout_passes" in pltpu.CompilerParams.__dataclass_fields__:
        cp = dataclasses.replace(cp, needs_layout_passes=False)
    return cp


def _tc1_body(y_ref, wu_ref, sx_ref, wb_ref, t_ref):
    t_ref[0:_N1, :] = jnp.dot(y_ref[...], wu_ref[...],
                              preferred_element_type=jnp.float32)
    t_ref[_N1:_ZONE1, :] = jnp.zeros((_ZONE1 - _N1, _D), jnp.float32)
    t_ref[_ZONE1:_ZONE1 + _N2, :] = jnp.dot(sx_ref[...], wb_ref[...],
                                            preferred_element_type=jnp.float32)
    t_ref[_ZONE1 + _N2:_TROWS, :] = jnp.zeros((_TROWS - _ZONE1 - _N2, _D),
                                              jnp.float32)


def _tc2_body(degp_ref, dis2_ref):
    degsum = degp_ref[0, :] + degp_ref[1, :]
    idx = lax.broadcasted_iota(jnp.int32, (_TROWS,), 0)
    dis2_ref[...] = jnp.where(idx < _ZONE1,
                              lax.rsqrt(jnp.maximum(degsum, 1e-30)),
                              jnp.float32(1.0))


def _tc3_body(accp_ref, x_ref):
    x_ref[...] = accp_ref[0, 0:_N1, :] + accp_ref[1, 0:_N1, :]


def _sc_deg(sidx3, w3, zdeg):
    mesh = plsc.VectorSubcoreMesh(core_axis_name="c", subcore_axis_name="s")

    @functools.partial(
        pl.kernel,
        out_type=jax.ShapeDtypeStruct((_NC * _TROWS,), jnp.float32),
        mesh=mesh,
        scratch_types=[
            pltpu.VMEM((_NBLK, _BLK), jnp.int32),
            pltpu.VMEM((_NBLK, _BLK), jnp.float32),
            pltpu.VMEM_SHARED((_TROWS,), jnp.float32),
        ],
    )
    def k(sidx_hbm, w_hbm, z_hbm, out_hbm, idx_v, w_v, acc_sh):
        c = lax.axis_index("c")
        s = lax.axis_index("s")
        wid = c * _NS + s
        pltpu.sync_copy(sidx_hbm.at[wid], idx_v)
        pltpu.sync_copy(w_hbm.at[wid], w_v)
        pltpu.sync_copy(z_hbm.at[pl.ds(s * _DEGC, _DEGC)],
                        acc_sh.at[pl.ds(s * _DEGC, _DEGC)])
        plsc.subcore_barrier()

        @pl.loop(0, _NBLK)
        def _(j):
            pltpu.sync_copy(w_v.at[j], acc_sh.at[idx_v.at[j]], add=True)

        plsc.subcore_barrier()
        pltpu.sync_copy(acc_sh.at[pl.ds(s * _DEGC, _DEGC)],
                        out_hbm.at[pl.ds(c * _TROWS + s * _DEGC, _DEGC)])

    return k(sidx3, w3, zdeg)


def _sc_feat(src4, dst4, sidx4, w4, t_hbm_arr, dis2_arr, zfeat):
    mesh = plsc.VectorSubcoreMesh(core_axis_name="c", subcore_axis_name="s")

    @functools.partial(
        pl.kernel,
        out_type=jax.ShapeDtypeStruct((_NC, _ACCP, _D), jnp.float32),
        mesh=mesh,
        scratch_types=[
            pltpu.VMEM((1, _CEDGE), jnp.int32),
            pltpu.VMEM((_CB, _BLK), jnp.int32),
            pltpu.VMEM((1, _CEDGE), jnp.int32),
            pltpu.VMEM((1, _CEDGE), jnp.float32),
            pltpu.VMEM((_TROWS,), jnp.float32),
            pltpu.VMEM((_BLK, _D), jnp.float32),
            pltpu.VMEM_SHARED((_ACCP, _D), jnp.float32),
        ],
        compiler_params=_sc_compiler_params(),
    )
    def k(src_hbm, dst_hbm, sidx_hbm, w_hbm, t_hbm, dis2_hbm, z_hbm, out_hbm,
          src_v, dst_v, sidx_v, w_v, dis2_v, gbuf, acc_sh):
        c = lax.axis_index("c")
        s = lax.axis_index("s")
        wid = c * _NS + s
        pltpu.sync_copy(dis2_hbm, dis2_v)
        pltpu.sync_copy(z_hbm.at[pl.ds(s * _ACCC, _ACCC)],
                        acc_sh.at[pl.ds(s * _ACCC, _ACCC)])
        plsc.subcore_barrier()

        zero16 = jnp.zeros((_L,), jnp.int32)

        @pl.loop(0, _NCHUNK)
        def _(ch):
            pltpu.sync_copy(src_hbm.at[wid, ch], src_v)
            pltpu.sync_copy(dst_hbm.at[wid, ch], dst_v)
            pltpu.sync_copy(sidx_hbm.at[wid, ch], sidx_v)
            pltpu.sync_copy(w_hbm.at[wid, ch], w_v)

            @pl.loop(0, _CEDGE, step=_L)
            def _(i):
                sl = (0, pl.ds(i, _L))
                a = plsc.load_gather(dis2_v, [sidx_v[sl]])
                b = plsc.load_gather(dis2_v, [src_v[sl]])
                w_v[sl] = w_v[sl] * a * b

            @pl.loop(0, _CB)
            def _(j):
                pltpu.sync_copy(
                    t_hbm.at[src_v.at[0, pl.ds(j * _BLK, _BLK)]], gbuf)

                @pl.loop(0, _BLK)
                def _(e):
                    cvec = plsc.load_gather(
                        w_v,
                        [zero16, jnp.full((_L,), j * _BLK + e, jnp.int32)])
                    for k8 in range(_D // _L):
                        sl = (e, pl.ds(k8 * _L, _L))
                        gbuf[sl] = gbuf[sl] * cvec

                pltpu.sync_copy(gbuf, acc_sh.at[dst_v.at[j]], add=True)

        plsc.subcore_barrier()
        pltpu.sync_copy(acc_sh.at[pl.ds(s * _ACCC, _ACCC)],
                        out_hbm.at[c, pl.ds(s * _ACCC, _ACCC)])

    return k(src4, dst4, sidx4, w4, t_hbm_arr, dis2_arr, zfeat)


def kernel(sup_x, y, edge_index, edge_weight, assign_index, assign_weight,
           W_bi, W_uni):
    E = edge_index.shape[1]
    A = assign_index.shape[1]
    row = edge_index[0]
    col = edge_index[1]
    asrc = assign_index[0]
    adst = assign_index[1]
    npad = _ETOT - (E + A + _N1)
    nid = jnp.arange(_N1, dtype=jnp.int32)
    padv = (jnp.arange(npad, dtype=jnp.int32) * 97) % _N1

    src_all = jnp.concatenate([row, asrc + _ZONE1, nid, padv])
    dst_all = jnp.concatenate([col, adst, nid, padv])
    sidx_all = jnp.concatenate(
        [col, _ZONE1 + (jnp.arange(A, dtype=jnp.int32) % (_TROWS - _ZONE1)),
         nid, padv])
    w_all = jnp.concatenate(
        [edge_weight, assign_weight, jnp.full((_N1,), 2.0, jnp.float32),
         jnp.zeros((npad,), jnp.float32)])

    sidx3 = sidx_all.reshape(_NW, _NBLK, _BLK)
    w3 = w_all.reshape(_NW, _NBLK, _BLK)
    dst4 = dst_all.reshape(_NW, _NCHUNK, _CB, _BLK)
    src4 = src_all.reshape(_NW, _NCHUNK, 1, _CEDGE)
    sidx4 = sidx_all.reshape(_NW, _NCHUNK, 1, _CEDGE)
    w4 = w_all.reshape(_NW, _NCHUNK, 1, _CEDGE)

    zdeg = jnp.zeros((_TROWS,), jnp.float32)
    zfeat = jnp.zeros((_ACCP, _D), jnp.float32)

    t_arr = pl.pallas_call(
        _tc1_body,
        out_shape=jax.ShapeDtypeStruct((_TROWS, _D), jnp.float32),
    )(y, W_uni, sup_x, W_bi)

    degp = _sc_deg(sidx3, w3, zdeg).reshape(_NC, _TROWS)

    dis2 = pl.pallas_call(
        _tc2_body,
        out_shape=jax.ShapeDtypeStruct((_TROWS,), jnp.float32),
    )(degp)

    accp = _sc_feat(src4, dst4, sidx4, w4, t_arr, dis2, zfeat)

    x = pl.pallas_call(
        _tc3_body,
        out_shape=jax.ShapeDtypeStruct((_N1, _D), jnp.float32),
    )(accp)

    return (x, edge_index, edge_weight)
